# baseline (speedup 1.0000x reference)
import jax
import jax.numpy as jnp
from jax.experimental import pallas as pl
from jax.experimental.pallas import tpu as pltpu

_HEADS = [("hm", 3), ("wh", 2), ("hps", 18), ("rot", 8), ("dim", 3),
          ("prob", 1), ("reg", 2), ("hm_hp", 9), ("hp_offset", 2)]
_B, _CIN, _CMID, _H, _W = 2, 256, 64, 96, 320
_NH = len(_HEADS)
_CM = _NH * _CMID
_COUT = sum(c for _, c in _HEADS)

_TH = 16
_NBH = _H // _TH
_MF = _TH * _W
_PADL = 448
_PADR = 448
_LSLAB = 6016


def _conv_body(x_hbm, *args):
    w1hs = args[0:_NH]
    b1hs = args[_NH:2 * _NH]
    w2hs = args[2 * _NH:3 * _NH]
    b2hs = args[3 * _NH:4 * _NH]
    (out_ref, ibuf, col_scr, w1_scr, b1_scr, w2_scr, b2_scr, sems) = \
        args[4 * _NH:]

    g = pl.program_id(0)
    nsteps = _B * _NBH

    @pl.when(g == 0)
    def _():
        whs = [w1hs[h][...].astype(jnp.bfloat16) for h in range(_NH)]
        i0 = jax.lax.broadcasted_iota(jnp.int32, (9 * _CIN, _CIN), 0)
        i1 = jax.lax.broadcasted_iota(jnp.int32, (9 * _CIN, _CIN), 1)
        for ky in range(3):
            for kx in range(3):
                sel = (i0 == i1 * 9 + (3 * ky + kx)).astype(jnp.bfloat16)
                for h in range(_NH):
                    tap = jax.lax.dot_general(
                        whs[h], sel, (((1,), (0,)), ((), ())),
                        preferred_element_type=jnp.float32)
                    w1_scr[ky, h * _CMID:(h + 1) * _CMID,
                           kx * _CIN:(kx + 1) * _CIN] = (
                        tap.astype(jnp.bfloat16))
        w2_scr[...] = jnp.zeros((_COUT, _CM), jnp.bfloat16)
        o = 0
        for h, (_, c) in enumerate(_HEADS):
            b1_scr[h * _CMID:(h + 1) * _CMID, :] = b1hs[h][...]
            w2_scr[o:o + c, h * _CMID:(h + 1) * _CMID] = (
                w2hs[h][...].astype(jnp.bfloat16))
            b2_scr[o:o + c, :] = b2hs[h][...]
            o += c

    def start_copy(slot, gg):
        pltpu.make_async_copy(
            x_hbm.at[gg // _NBH, :, pl.ds((gg % _NBH) * _TH * _W, _LSLAB)],
            ibuf.at[slot], sems.at[slot]).start()

    @pl.when(g == 0)
    def _():
        start_copy(0, 0)

    @pl.when(g + 1 < nsteps)
    def _():
        start_copy((g + 1) % 2, g + 1)

    slot = g % 2
    pltpu.make_async_copy(
        x_hbm.at[0, :, pl.ds(0, _LSLAB)], ibuf.at[slot], sems.at[slot]).wait()

    wcol = jax.lax.broadcasted_iota(jnp.int32, (1, _MF), 1) % _W
    acc = None
    for ky in range(3):
        for kx in range(3):
            off = 128 + ky * _W + kx - 1
            a = ibuf[slot, :, off:off + _MF]
            if kx == 0:
                a = jnp.where(wcol == 0, jnp.bfloat16(0), a)
            elif kx == 2:
                a = jnp.where(wcol == _W - 1, jnp.bfloat16(0), a)
            col_scr[kx * _CIN:(kx + 1) * _CIN, :] = a
        d = jax.lax.dot_general(
            w1_scr[ky], col_scr[...], (((1,), (0,)), ((), ())),
            preferred_element_type=jnp.float32)
        acc = d if acc is None else acc + d
    mid = jnp.maximum(acc + b1_scr[...], 0.0).astype(jnp.bfloat16)
    out_ref[0] = jax.lax.dot_general(
        w2_scr[...], mid, (((1,), (0,)), ((), ())),
        preferred_element_type=jnp.float32) + b2_scr[...]


def kernel(x, hm_W1, hm_b1, hm_W2, hm_b2, wh_W1, wh_b1, wh_W2, wh_b2,
           hps_W1, hps_b1, hps_W2, hps_b2, rot_W1, rot_b1, rot_W2, rot_b2,
           dim_W1, dim_b1, dim_W2, dim_b2, prob_W1, prob_b1, prob_W2, prob_b2,
           reg_W1, reg_b1, reg_W2, reg_b2, hm_hp_W1, hm_hp_b1, hm_hp_W2,
           hm_hp_b2, hp_offset_W1, hp_offset_b1, hp_offset_W2, hp_offset_b2):
    params = dict(locals())
    w1s = [params[n + "_W1"].reshape(_CMID, _CIN * 9) for n, _ in _HEADS]
    b1s = [params[n + "_b1"].reshape(_CMID, 1) for n, _ in _HEADS]
    w2s = [params[n + "_W2"].reshape(-1, _CMID) for n, _ in _HEADS]
    b2s = [params[n + "_b2"].reshape(-1, 1) for n, _ in _HEADS]

    xp = jnp.pad(x.astype(jnp.bfloat16).reshape(_B, _CIN, _H * _W),
                 ((0, 0), (0, 0), (_PADL, _PADR)))

    def _const_spec(a):
        shape = a.shape
        return pl.BlockSpec(shape, lambda g: tuple(0 for _ in shape))

    grid = (_B * _NBH,)
    out = pl.pallas_call(
        _conv_body,
        grid=grid,
        in_specs=[pl.BlockSpec(memory_space=pltpu.MemorySpace.HBM)]
                 + [_const_spec(a) for a in w1s + b1s + w2s + b2s],
        out_specs=pl.BlockSpec((1, _COUT, _MF),
                               lambda g: (g // _NBH, 0, g % _NBH)),
        out_shape=jax.ShapeDtypeStruct((_B, _COUT, _H * _W), jnp.float32),
        scratch_shapes=[
            pltpu.VMEM((2, _CIN, _LSLAB), jnp.bfloat16),
            pltpu.VMEM((3 * _CIN, _MF), jnp.bfloat16),
            pltpu.VMEM((3, _CM, 3 * _CIN), jnp.bfloat16),
            pltpu.VMEM((_CM, 1), jnp.float32),
            pltpu.VMEM((_COUT, _CM), jnp.bfloat16),
            pltpu.VMEM((_COUT, 1), jnp.float32),
            pltpu.SemaphoreType.DMA((2,)),
        ],
    )(xp, *w1s, *b1s, *w2s, *b2s)

    return out.reshape(_B, _COUT, _H, _W)


# speedup vs baseline: 1.1435x; 1.1435x over previous
import jax
import jax.numpy as jnp
from jax.experimental import pallas as pl
from jax.experimental.pallas import tpu as pltpu

_HEADS = [("hm", 3), ("wh", 2), ("hps", 18), ("rot", 8), ("dim", 3),
          ("prob", 1), ("reg", 2), ("hm_hp", 9), ("hp_offset", 2)]
_B, _CIN, _CMID, _H, _W = 2, 256, 64, 96, 320
_NH = len(_HEADS)
_CM = _NH * _CMID
_COUT = sum(c for _, c in _HEADS)

_TH = 16
_NBH = _H // _TH
_MF = _TH * _W
_S = 384
_LSLAB = 5888
_LS_LAST = _H * _W - ((_NBH - 1) * _MF - _S)


def _conv_body(x_hbm, w1_ref, b1_ref, w2c_ref, b2_ref, out_ref,
               ibuf, w2_scr, col_scr, sems):
    g = pl.program_id(0)
    nsteps = _B * _NBH

    @pl.when(g == 0)
    def _():
        w2_scr[...] = jnp.zeros((_COUT, _CM), jnp.bfloat16)
        o = 0
        for h, (_, c) in enumerate(_HEADS):
            w2_scr[o:o + c, h * _CMID:(h + 1) * _CMID] = w2c_ref[o:o + c, :]
            o += c

    def start_copy(slot, gg):
        b = gg // _NBH
        band = gg % _NBH

        @pl.when(band == 0)
        def _():
            pltpu.make_async_copy(
                x_hbm.at[b, :, pl.ds(0, _LSLAB - _S)],
                ibuf.at[slot, :, pl.ds(_S, _LSLAB - _S)],
                sems.at[slot]).start()
            ibuf[slot, :, 0:_S] = jnp.zeros((_CIN, _S), jnp.float32)

        @pl.when(jnp.logical_and(band > 0, band < _NBH - 1))
        def _():
            pltpu.make_async_copy(
                x_hbm.at[b, :, pl.ds(band * _MF - _S, _LSLAB)],
                ibuf.at[slot], sems.at[slot]).start()

        @pl.when(band == _NBH - 1)
        def _():
            pltpu.make_async_copy(
                x_hbm.at[b, :, pl.ds(band * _MF - _S, _LS_LAST)],
                ibuf.at[slot, :, pl.ds(0, _LS_LAST)],
                sems.at[slot]).start()
            ibuf[slot, :, _LS_LAST:_LSLAB] = jnp.zeros(
                (_CIN, _LSLAB - _LS_LAST), jnp.float32)

    @pl.when(g == 0)
    def _():
        start_copy(0, 0)

    @pl.when(g + 1 < nsteps)
    def _():
        start_copy((g + 1) % 2, g + 1)

    slot = g % 2
    band = g % _NBH

    @pl.when(band == 0)
    def _():
        pltpu.make_async_copy(
            x_hbm.at[0, :, pl.ds(0, _LSLAB - _S)],
            ibuf.at[slot, :, pl.ds(_S, _LSLAB - _S)], sems.at[slot]).wait()

    @pl.when(jnp.logical_and(band > 0, band < _NBH - 1))
    def _():
        pltpu.make_async_copy(
            x_hbm.at[0, :, pl.ds(0, _LSLAB)],
            ibuf.at[slot], sems.at[slot]).wait()

    @pl.when(band == _NBH - 1)
    def _():
        pltpu.make_async_copy(
            x_hbm.at[0, :, pl.ds(0, _LS_LAST)],
            ibuf.at[slot, :, pl.ds(0, _LS_LAST)], sems.at[slot]).wait()

    sb = ibuf[slot].astype(jnp.bfloat16)

    wcol = jax.lax.broadcasted_iota(jnp.int32, (1, _MF), 1) % _W
    acc = None
    for ky in range(3):
        for kx in range(3):
            off = _S - _W - 1 + ky * _W + kx
            a = sb[:, off:off + _MF]
            if kx == 0:
                a = jnp.where(wcol == 0, jnp.bfloat16(0), a)
            elif kx == 2:
                a = jnp.where(wcol == _W - 1, jnp.bfloat16(0), a)
            col_scr[kx * _CIN:(kx + 1) * _CIN, :] = a
        d = jax.lax.dot_general(
            w1_ref[ky], col_scr[...], (((1,), (0,)), ((), ())),
            preferred_element_type=jnp.float32)
        acc = d if acc is None else acc + d
    mid = jnp.maximum(acc + b1_ref[...], 0.0).astype(jnp.bfloat16)
    out_ref[0] = jax.lax.dot_general(
        w2_scr[...], mid, (((1,), (0,)), ((), ())),
        preferred_element_type=jnp.float32) + b2_ref[...]


def kernel(x, hm_W1, hm_b1, hm_W2, hm_b2, wh_W1, wh_b1, wh_W2, wh_b2,
           hps_W1, hps_b1, hps_W2, hps_b2, rot_W1, rot_b1, rot_W2, rot_b2,
           dim_W1, dim_b1, dim_W2, dim_b2, prob_W1, prob_b1, prob_W2, prob_b2,
           reg_W1, reg_b1, reg_W2, reg_b2, hm_hp_W1, hm_hp_b1, hm_hp_W2,
           hm_hp_b2, hp_offset_W1, hp_offset_b1, hp_offset_W2, hp_offset_b2):
    params = dict(locals())
    w1s = [params[n + "_W1"] for n, _ in _HEADS]
    b1s = [params[n + "_b1"] for n, _ in _HEADS]
    w2s = [params[n + "_W2"] for n, _ in _HEADS]
    b2s = [params[n + "_b2"] for n, _ in _HEADS]

    w1 = jnp.concatenate(w1s, axis=0).transpose(2, 0, 3, 1)
    w1 = w1.reshape(3, _CM, 3 * _CIN).astype(jnp.bfloat16)
    b1 = jnp.concatenate(b1s, axis=0).reshape(_CM, 1)
    w2 = jnp.concatenate(
        [w.reshape(-1, _CMID) for w in w2s], axis=0).astype(jnp.bfloat16)
    b2 = jnp.concatenate(b2s, axis=0).reshape(_COUT, 1)

    xp = x.reshape(_B, _CIN, _H * _W)

    grid = (_B * _NBH,)
    out = pl.pallas_call(
        _conv_body,
        grid=grid,
        in_specs=[
            pl.BlockSpec(memory_space=pltpu.MemorySpace.HBM),
            pl.BlockSpec((3, _CM, 3 * _CIN), lambda g: (0, 0, 0)),
            pl.BlockSpec((_CM, 1), lambda g: (0, 0)),
            pl.BlockSpec((_COUT, _CMID), lambda g: (0, 0)),
            pl.BlockSpec((_COUT, 1), lambda g: (0, 0)),
        ],
        out_specs=pl.BlockSpec((1, _COUT, _MF),
                               lambda g: (g // _NBH, 0, g % _NBH)),
        out_shape=jax.ShapeDtypeStruct((_B, _COUT, _H * _W), jnp.float32),
        scratch_shapes=[
            pltpu.VMEM((2, _CIN, _LSLAB), jnp.float32),
            pltpu.VMEM((_COUT, _CM), jnp.bfloat16),
            pltpu.VMEM((3 * _CIN, _MF), jnp.bfloat16),
            pltpu.SemaphoreType.DMA((2,)),
        ],
    )(xp, w1, b1, w2, b2)

    return out.reshape(_B, _COUT, _H, _W)
